# Initial kernel scaffold; baseline (speedup 1.0000x reference)
import jax
import jax.numpy as jnp
from jax.experimental import pallas as pl


def kernel(z, u, x, edge_index, batch, batch_size, prev_h, Wp, bp, Wg, bg, W_xz, b_xz, W_hz, b_hz, W_xr, b_xr, W_hr, b_hr, W_xh, b_xh, W_hh, b_hh):
    raise NotImplementedError("write your pallas kernel here")


# speedup vs baseline: 3.0895x; 3.0895x over previous
import functools

import jax
import jax.numpy as jnp
from jax import lax
from jax.experimental import pallas as pl
from jax.experimental.pallas import tpu as pltpu
from jax.experimental.pallas import tpu_sc as plsc

_NC = 2
_NS = 16
_LANES = 16


def _make_sc_segsum(num_chunks, chunk, d, num_seg):
  nw = _NC * _NS
  chunks_per_tile = (num_chunks + nw - 1) // nw
  mesh = plsc.VectorSubcoreMesh(core_axis_name="c", subcore_axis_name="s",
                                num_cores=_NC, num_subcores=_NS)

  @functools.partial(
      pl.kernel,
      out_type=jax.ShapeDtypeStruct((_NC, num_seg, d), jnp.float32),
      mesh=mesh,
      scratch_types=[
          pltpu.VMEM((chunk, d), jnp.float32),
          pltpu.VMEM((chunks_per_tile, chunk), jnp.int32),
          pltpu.VMEM((num_seg, d), jnp.float32),
          pltpu.VMEM_SHARED((num_seg, d), jnp.float32),
      ],
  )
  def sc_segsum(z3, b2, sums_out, rows_v, idx_v, zsum_v, acc_sh):
    cid = lax.axis_index("c")
    sid = lax.axis_index("s")
    wid = cid * _NS + sid

    @pl.when(sid == 0)
    def _init():
      zeros = jnp.zeros((_LANES,), jnp.float32)

      def fill_zsum(i, _):
        for k in range(d // _LANES):
          zsum_v[i, pl.ds(k * _LANES, _LANES)] = zeros
        return 0

      lax.fori_loop(0, num_seg, fill_zsum, 0)
      pltpu.sync_copy(zsum_v, acc_sh)

    plsc.subcore_barrier()

    for j in range(chunks_per_tile):
      c = j * nw + wid

      @pl.when(c < num_chunks)
      def _do_chunk():
        pltpu.sync_copy(z3.at[c], rows_v)
        pltpu.sync_copy(b2.at[c], idx_v.at[j])
        pltpu.sync_copy(rows_v, acc_sh.at[idx_v.at[j]], add=True)

    plsc.subcore_barrier()

    @pl.when(sid == 0)
    def _writeout():
      pltpu.sync_copy(acc_sh, sums_out.at[cid])

  return sc_segsum


def _gru_body(z_ref, x_ref, h_ref, wp_ref, bp_ref, wx_ref, bx_ref,
              whzr_ref, bhzr_ref, whh_ref, bhh_ref, out_ref):
  f32 = jnp.float32
  db = whh_ref.shape[0]
  h = h_ref[...]
  xp = jnp.maximum(
      jnp.dot(x_ref[...], wp_ref[...], preferred_element_type=f32)
      + bp_ref[...], 0.0)
  gin = jnp.concatenate([z_ref[...], xp], axis=1)
  a = jnp.dot(gin, wx_ref[...], preferred_element_type=f32) + bx_ref[...]
  ah = jnp.dot(h, whzr_ref[...], preferred_element_type=f32) + bhzr_ref[...]
  zg = jax.nn.sigmoid(a[:, :db] + ah[:, :db])
  rg = jax.nn.sigmoid(a[:, db:2 * db] + ah[:, db:2 * db])
  ht = jnp.tanh(a[:, 2 * db:] +
                jnp.dot(rg * h, whh_ref[...], preferred_element_type=f32)
                + bhh_ref[...])
  out_ref[...] = zg * h + (1.0 - zg) * ht


def _make_fused_body(num_seg):

  def fused_body(s_ref, bp_ref, u_ref, wg_ref, bg_ref, out_ref):
    s = s_ref[0] + s_ref[1]
    bb = bp_ref[...]
    ii = lax.broadcasted_iota(jnp.int32, (num_seg, 1, 1), 0)
    cmp = (bb[None, :, :] == ii).astype(jnp.float32)
    cnt = jnp.sum(jnp.sum(cmp, axis=2), axis=1, keepdims=True)
    ge = s / jnp.maximum(cnt, 1.0)
    glob = jnp.maximum(
        jnp.dot(u_ref[...], wg_ref[...], preferred_element_type=jnp.float32)
        + bg_ref[...], 0.0)
    out_ref[...] = jnp.concatenate([ge, glob], axis=1)

  return fused_body


def kernel(z, u, x, edge_index, batch, batch_size, prev_h, Wp, bp, Wg, bg,
           W_xz, b_xz, W_hz, b_hz, W_xr, b_xr, W_hr, b_hr, W_xh, b_xh,
           W_hh, b_hh):
  n, db = z.shape
  df = x.shape[1]
  dp = Wp.shape[1]
  b = u.shape[0]
  gin_d = db + dp

  chunk = 1
  for c in range(min(128, n), 0, -1):
    if n % c == 0:
      chunk = c
      break
  num_chunks = n // chunk
  z3 = z.reshape(num_chunks, chunk, db)
  batch = batch.astype(jnp.int32)
  b2 = batch.reshape(num_chunks, chunk)
  sums = _make_sc_segsum(num_chunks, chunk, db, b)(z3, b2)

  brows = (n + 127) // 128
  bpad = jnp.full((brows * 128,), b, jnp.int32).at[:n].set(batch)
  bpad = bpad.reshape(brows, 128)

  wx_all = jnp.concatenate([W_xz, W_xr, W_xh], axis=1)
  bx_all = jnp.concatenate([b_xz, b_xr, b_xh]).reshape(1, 3 * db)
  wh_zr = jnp.concatenate([W_hz, W_hr], axis=1)
  bh_zr = jnp.concatenate([b_hz, b_hr]).reshape(1, 2 * db)
  bp2 = bp.reshape(1, dp)
  bhh2 = b_hh.reshape(1, db)

  blk = 1000
  grid = (n // blk,)
  row_spec = lambda width: pl.BlockSpec((blk, width), lambda i: (i, 0))
  full = lambda s: pl.BlockSpec(s, lambda i: (0,) * len(s))
  h_new = pl.pallas_call(
      _gru_body,
      grid=grid,
      in_specs=[
          row_spec(db), row_spec(df), row_spec(db),
          full((df, dp)), full((1, dp)),
          full((gin_d, 3 * db)), full((1, 3 * db)),
          full((db, 2 * db)), full((1, 2 * db)),
          full((db, db)), full((1, db)),
      ],
      out_specs=row_spec(db),
      out_shape=jax.ShapeDtypeStruct((n, db), jnp.float32),
  )(z, x, prev_h, Wp, bp2, wx_all, bx_all, wh_zr, bh_zr, W_hh, bhh2)

  go = Wg.shape[1]
  fused = pl.pallas_call(
      _make_fused_body(b),
      out_shape=jax.ShapeDtypeStruct((b, db + go), jnp.float32),
  )(sums, bpad, u, Wg, bg.reshape(1, go))

  return (fused, h_new)


# speedup vs baseline: 3.6385x; 1.1777x over previous
import functools

import jax
import jax.numpy as jnp
from jax import lax
from jax.experimental import pallas as pl
from jax.experimental.pallas import tpu as pltpu
from jax.experimental.pallas import tpu_sc as plsc

_NC = 2
_NS = 16
_LANES = 16


def _make_sc_segsum(n, chunk, d, num_seg):
  nw = _NC * _NS
  num_chunks = n // chunk
  chunks_per_tile = (num_chunks + nw - 1) // nw
  mesh = plsc.VectorSubcoreMesh(core_axis_name="c", subcore_axis_name="s",
                                num_cores=_NC, num_subcores=_NS)

  @functools.partial(
      pl.kernel,
      out_type=[
          jax.ShapeDtypeStruct((_NC, num_seg, d), jnp.float32),
          jax.ShapeDtypeStruct((_NC, num_seg, d), jnp.float32),
      ],
      mesh=mesh,
      scratch_types=[
          pltpu.VMEM((chunk, d), jnp.float32),
          pltpu.VMEM((chunks_per_tile, chunk), jnp.int32),
          pltpu.VMEM((chunk, d), jnp.float32),
          pltpu.VMEM((num_seg, d), jnp.float32),
          pltpu.VMEM_SHARED((num_seg, d), jnp.float32),
          pltpu.VMEM_SHARED((num_seg, d), jnp.float32),
      ],
  )
  def sc_segsum(z_hbm, b_hbm, sums_out, cnts_out, rows_v, idx_v, ones_v,
                zeros_v, acc_sh, cnt_sh):
    cid = lax.axis_index("c")
    sid = lax.axis_index("s")
    wid = cid * _NS + sid

    ones = jnp.ones((_LANES,), jnp.float32)

    def fill_ones(i, _):
      for k in range(d // _LANES):
        ones_v[i, pl.ds(k * _LANES, _LANES)] = ones
      return 0

    lax.fori_loop(0, chunk, fill_ones, 0)

    @pl.when(sid == 0)
    def _init():
      zeros = jnp.zeros((_LANES,), jnp.float32)

      def fill_zeros(i, _):
        for k in range(d // _LANES):
          zeros_v[i, pl.ds(k * _LANES, _LANES)] = zeros
        return 0

      lax.fori_loop(0, num_seg, fill_zeros, 0)
      pltpu.sync_copy(zeros_v, acc_sh)
      pltpu.sync_copy(zeros_v, cnt_sh)

    plsc.subcore_barrier()

    for j in range(chunks_per_tile):
      c = j * nw + wid

      @pl.when(c < num_chunks)
      def _do_chunk():
        base = c * chunk
        pltpu.sync_copy(z_hbm.at[pl.ds(base, chunk), :], rows_v)
        pltpu.sync_copy(b_hbm.at[pl.ds(base, chunk)], idx_v.at[j])
        pltpu.sync_copy(rows_v, acc_sh.at[idx_v.at[j]], add=True)
        pltpu.sync_copy(ones_v, cnt_sh.at[idx_v.at[j]], add=True)

    plsc.subcore_barrier()

    @pl.when(sid == 0)
    def _writeout():
      pltpu.sync_copy(acc_sh, sums_out.at[cid])
      pltpu.sync_copy(cnt_sh, cnts_out.at[cid])

  return sc_segsum


def _gru_body(z_ref, x_ref, h_ref, wp_ref, bp_ref, wx_ref, bx_ref,
              whzr_ref, bhzr_ref, whh_ref, bhh_ref, out_ref):
  f32 = jnp.float32
  db = whh_ref.shape[0]
  h = h_ref[...]
  xp = jnp.maximum(
      jnp.dot(x_ref[...], wp_ref[...], preferred_element_type=f32)
      + bp_ref[...], 0.0)
  gin = jnp.concatenate([z_ref[...], xp], axis=1)
  a = jnp.dot(gin, wx_ref[...], preferred_element_type=f32) + bx_ref[...]
  ah = jnp.dot(h, whzr_ref[...], preferred_element_type=f32) + bhzr_ref[...]
  zg = jax.nn.sigmoid(a[:, :db] + ah[:, :db])
  rg = jax.nn.sigmoid(a[:, db:2 * db] + ah[:, db:2 * db])
  ht = jnp.tanh(a[:, 2 * db:] +
                jnp.dot(rg * h, whh_ref[...], preferred_element_type=f32)
                + bhh_ref[...])
  out_ref[...] = zg * h + (1.0 - zg) * ht


def _fused_body(s_ref, c_ref, u_ref, wg_ref, bg_ref, out_ref):
  s = s_ref[0] + s_ref[1]
  cnt = c_ref[0, :, 0:1] + c_ref[1, :, 0:1]
  ge = s / jnp.maximum(cnt, 1.0)
  glob = jnp.maximum(
      jnp.dot(u_ref[...], wg_ref[...], preferred_element_type=jnp.float32)
      + bg_ref[...], 0.0)
  out_ref[...] = jnp.concatenate([ge, glob], axis=1)


def kernel(z, u, x, edge_index, batch, batch_size, prev_h, Wp, bp, Wg, bg,
           W_xz, b_xz, W_hz, b_hz, W_xr, b_xr, W_hr, b_hr, W_xh, b_xh,
           W_hh, b_hh):
  n, db = z.shape
  df = x.shape[1]
  dp = Wp.shape[1]
  b = u.shape[0]
  gin_d = db + dp

  chunk = 1
  for c in range(min(128, n), 0, -1):
    if n % c == 0 and c % 8 == 0:
      chunk = c
      break
  batch = batch.astype(jnp.int32)
  sums, cnts = _make_sc_segsum(n, chunk, db, b)(z, batch)

  wx_all = jnp.concatenate([W_xz, W_xr, W_xh], axis=1)
  bx_all = jnp.concatenate([b_xz, b_xr, b_xh]).reshape(1, 3 * db)
  wh_zr = jnp.concatenate([W_hz, W_hr], axis=1)
  bh_zr = jnp.concatenate([b_hz, b_hr]).reshape(1, 2 * db)
  bp2 = bp.reshape(1, dp)
  bhh2 = b_hh.reshape(1, db)

  blk = 1000
  grid = (n // blk,)
  row_spec = lambda width: pl.BlockSpec((blk, width), lambda i: (i, 0))
  full = lambda s: pl.BlockSpec(s, lambda i: (0,) * len(s))
  h_new = pl.pallas_call(
      _gru_body,
      grid=grid,
      in_specs=[
          row_spec(db), row_spec(df), row_spec(db),
          full((df, dp)), full((1, dp)),
          full((gin_d, 3 * db)), full((1, 3 * db)),
          full((db, 2 * db)), full((1, 2 * db)),
          full((db, db)), full((1, db)),
      ],
      out_specs=row_spec(db),
      out_shape=jax.ShapeDtypeStruct((n, db), jnp.float32),
  )(z, x, prev_h, Wp, bp2, wx_all, bx_all, wh_zr, bh_zr, W_hh, bhh2)

  go = Wg.shape[1]
  fused = pl.pallas_call(
      _fused_body,
      out_shape=jax.ShapeDtypeStruct((b, db + go), jnp.float32),
  )(sums, cnts, u, Wg, bg.reshape(1, go))

  return (fused, h_new)
